# baseline (device time: 172207 ns/iter reference)
import jax
import jax.numpy as jnp
from jax import lax
from jax.experimental import pallas as pl
from jax.experimental.pallas import tpu as pltpu

N_DEV = 4
N_HOP = N_DEV - 1


def kernel(x, w_mat):
    w = w_mat.astype(jnp.bfloat16)
    m_per, k = x.shape
    _, n = w.shape
    half = m_per // 2

    def body(x_hbm, w_ref, out_ref, stage, x_bf, cw_ref, ccw_ref,
             cw_send, cw_recv, ccw_send, ccw_recv, copy_sems):
        my = lax.axis_index("i")
        left = lax.rem(my + N_DEV - 1, N_DEV)
        right = lax.rem(my + 1, N_DEV)

        barrier_sem = pltpu.get_barrier_semaphore()
        for nbr in (left, right):
            pl.semaphore_signal(
                barrier_sem, inc=1,
                device_id=(nbr,), device_id_type=pl.DeviceIdType.MESH,
            )

        cp0 = pltpu.make_async_copy(
            x_hbm.at[pl.ds(0, half)], stage, copy_sems.at[0])
        cp0.start()
        cp0.wait()
        x_bf[pl.ds(0, half)] = stage[...].astype(jnp.bfloat16)
        cp1 = pltpu.make_async_copy(
            x_hbm.at[pl.ds(half, half)], stage, copy_sems.at[1])
        cp1.start()

        pl.semaphore_wait(barrier_sem, 2)

        def gelu_store(chunk, row_start, rows):
            y = lax.dot_general(
                chunk, w_ref[...],
                (((1,), (0,)), ((), ())),
                preferred_element_type=jnp.float32,
            )
            c = 0.7978845608028654
            g = 0.5 * y * (1.0 + jnp.tanh(c * (y + 0.044715 * y * y * y)))
            out_ref[pl.ds(row_start, rows), :] = g

        def send(src, dst_ref, sems_s, sems_r, h, nbr):
            r = pltpu.make_async_remote_copy(
                src_ref=src,
                dst_ref=dst_ref.at[h],
                send_sem=sems_s.at[h],
                recv_sem=sems_r.at[h],
                device_id=(nbr,),
                device_id_type=pl.DeviceIdType.MESH,
            )
            r.start()
            return r

        pending = []
        cw = send(x_bf.at[pl.ds(0, half)], cw_ref, cw_send, cw_recv, 0, right)
        cp1.wait()
        x_bf[pl.ds(half, half)] = stage[...].astype(jnp.bfloat16)
        ccw = send(x_bf.at[pl.ds(half, half)], ccw_ref, ccw_send, ccw_recv,
                   0, left)
        pending += [cw, ccw]
        gelu_store(x_bf[...], my * m_per, m_per)
        cw.wait_recv()
        ccw.wait_recv()

        for h in range(1, N_HOP):
            cw = send(cw_ref.at[h - 1], cw_ref, cw_send, cw_recv, h, right)
            ccw = send(ccw_ref.at[h - 1], ccw_ref, ccw_send, ccw_recv, h, left)
            pending += [cw, ccw]
            cw_origin = lax.rem(my + N_DEV - h, N_DEV)
            ccw_origin = lax.rem(my + h, N_DEV)
            gelu_store(cw_ref[h - 1], cw_origin * m_per, half)
            gelu_store(ccw_ref[h - 1], ccw_origin * m_per + half, half)
            cw.wait_recv()
            ccw.wait_recv()

        cw_origin = lax.rem(my + N_DEV - N_HOP, N_DEV)
        ccw_origin = lax.rem(my + N_HOP, N_DEV)
        gelu_store(cw_ref[N_HOP - 1], cw_origin * m_per, half)
        gelu_store(ccw_ref[N_HOP - 1], ccw_origin * m_per + half, half)

        for r in pending:
            r.wait_send()

    return pl.pallas_call(
        body,
        out_shape=jax.ShapeDtypeStruct((N_DEV * m_per, n), jnp.float32),
        in_specs=[
            pl.BlockSpec(memory_space=pl.ANY),
            pl.BlockSpec(memory_space=pltpu.VMEM),
        ],
        out_specs=pl.BlockSpec(memory_space=pltpu.VMEM),
        scratch_shapes=[
            pltpu.VMEM((half, k), jnp.float32),
            pltpu.VMEM((m_per, k), jnp.bfloat16),
            pltpu.VMEM((N_HOP, half, k), jnp.bfloat16),
            pltpu.VMEM((N_HOP, half, k), jnp.bfloat16),
            pltpu.SemaphoreType.DMA((N_HOP,)),
            pltpu.SemaphoreType.DMA((N_HOP,)),
            pltpu.SemaphoreType.DMA((N_HOP,)),
            pltpu.SemaphoreType.DMA((N_HOP,)),
            pltpu.SemaphoreType.DMA((2,)),
        ],
        compiler_params=pltpu.CompilerParams(
            collective_id=0,
            vmem_limit_bytes=100 * 1024 * 1024,
        ),
    )(x, w)


# device time: 159666 ns/iter; 1.0785x vs baseline; 1.0785x over previous
import jax
import jax.numpy as jnp
from jax import lax
from jax.experimental import pallas as pl
from jax.experimental.pallas import tpu as pltpu

N_DEV = 4
N_HOP = N_DEV - 1
SUB = 4


def kernel(x, w_mat):
    w = w_mat.astype(jnp.bfloat16)
    m_per, k = x.shape
    _, n = w.shape
    half = m_per // 2
    rows = half // SUB

    def body(x_hbm, w_ref, out_ref, stage, x_bf, cw_ref, ccw_ref,
             cw_send, cw_recv, ccw_send, ccw_recv, copy_sems):
        my = lax.axis_index("i")
        left = lax.rem(my + N_DEV - 1, N_DEV)
        right = lax.rem(my + 1, N_DEV)

        barrier_sem = pltpu.get_barrier_semaphore()
        for nbr in (left, right):
            pl.semaphore_signal(
                barrier_sem, inc=1,
                device_id=(nbr,), device_id_type=pl.DeviceIdType.MESH,
            )

        def x_row(j):
            return (j // 2) * rows + (half if j % 2 else 0)

        cps = [
            pltpu.make_async_copy(
                x_hbm.at[pl.ds(x_row(j), rows)],
                stage.at[j % 2],
                copy_sems.at[j],
            )
            for j in range(2 * SUB)
        ]
        cps[0].start()
        cps[1].start()

        pl.semaphore_wait(barrier_sem, 2)

        pending = []
        cw_rdmas = {}
        ccw_rdmas = {}

        def rsend(is_cw, h, s, src):
            ref, sems_s, sems_r, nbr = (
                (cw_ref, cw_send, cw_recv, right) if is_cw
                else (ccw_ref, ccw_send, ccw_recv, left)
            )
            r = pltpu.make_async_remote_copy(
                src_ref=src,
                dst_ref=ref.at[h].at[pl.ds(s * rows, rows)],
                send_sem=sems_s.at[h, s],
                recv_sem=sems_r.at[h, s],
                device_id=(nbr,),
                device_id_type=pl.DeviceIdType.MESH,
            )
            r.start()
            pending.append(r)
            (cw_rdmas if is_cw else ccw_rdmas)[(h, s)] = r

        def gelu_store(chunk, row_start, nrows):
            y = lax.dot_general(
                chunk, w_ref[...],
                (((1,), (0,)), ((), ())),
                preferred_element_type=jnp.float32,
            )
            c = 0.7978845608028654
            g = 0.5 * y * (1.0 + jnp.tanh(c * (y + 0.044715 * y * y * y)))
            out_ref[pl.ds(row_start, nrows), :] = g

        def cw_origin(j):
            return lax.rem(my + N_DEV - j - 1, N_DEV)

        def ccw_origin(j):
            return lax.rem(my + j + 1, N_DEV)

        for j in range(2 * SUB):
            cps[j].wait()
            r0 = x_row(j)
            x_bf[pl.ds(r0, rows)] = stage[j % 2].astype(jnp.bfloat16)
            if j + 2 < 2 * SUB:
                cps[j + 2].start()
            rsend(j % 2 == 0, 0, j // 2, x_bf.at[pl.ds(r0, rows)])

        gelu_store(x_bf[...], my * m_per, m_per)

        for h in range(1, N_HOP):
            for s in range(SUB):
                cw_rdmas[(h - 1, s)].wait_recv()
                rsend(True, h, s, cw_ref.at[h - 1].at[pl.ds(s * rows, rows)])
                ccw_rdmas[(h - 1, s)].wait_recv()
                rsend(False, h, s, ccw_ref.at[h - 1].at[pl.ds(s * rows, rows)])
            gelu_store(cw_ref[h - 1], cw_origin(h - 1) * m_per, half)
            gelu_store(ccw_ref[h - 1], ccw_origin(h - 1) * m_per + half, half)

        co = cw_origin(N_HOP - 1) * m_per
        xo = ccw_origin(N_HOP - 1) * m_per + half
        for s in range(SUB):
            cw_rdmas[(N_HOP - 1, s)].wait_recv()
            gelu_store(cw_ref[N_HOP - 1, pl.ds(s * rows, rows)],
                       co + s * rows, rows)
            ccw_rdmas[(N_HOP - 1, s)].wait_recv()
            gelu_store(ccw_ref[N_HOP - 1, pl.ds(s * rows, rows)],
                       xo + s * rows, rows)

        for r in pending:
            r.wait_send()

    return pl.pallas_call(
        body,
        out_shape=jax.ShapeDtypeStruct((N_DEV * m_per, n), jnp.float32),
        in_specs=[
            pl.BlockSpec(memory_space=pl.ANY),
            pl.BlockSpec(memory_space=pltpu.VMEM),
        ],
        out_specs=pl.BlockSpec(memory_space=pltpu.VMEM),
        scratch_shapes=[
            pltpu.VMEM((2, rows, k), jnp.float32),
            pltpu.VMEM((m_per, k), jnp.bfloat16),
            pltpu.VMEM((N_HOP, half, k), jnp.bfloat16),
            pltpu.VMEM((N_HOP, half, k), jnp.bfloat16),
            pltpu.SemaphoreType.DMA((N_HOP, SUB)),
            pltpu.SemaphoreType.DMA((N_HOP, SUB)),
            pltpu.SemaphoreType.DMA((N_HOP, SUB)),
            pltpu.SemaphoreType.DMA((N_HOP, SUB)),
            pltpu.SemaphoreType.DMA((2 * SUB,)),
        ],
        compiler_params=pltpu.CompilerParams(
            collective_id=0,
            vmem_limit_bytes=100 * 1024 * 1024,
        ),
    )(x, w)


# device time: 154459 ns/iter; 1.1149x vs baseline; 1.0337x over previous
import jax
import jax.numpy as jnp
from jax import lax
from jax.experimental import pallas as pl
from jax.experimental.pallas import tpu as pltpu

N_DEV = 4
N_HOP = N_DEV - 1
SUB = 4


def kernel(x, w_mat):
    m_per, k = x.shape
    _, n = w_mat.shape
    half = m_per // 2
    rows = half // SUB

    def body(x_hbm, w_hbm, out_ref, stage, w_f32, w_ref, x_bf, cw_ref,
             ccw_ref, cw_send, cw_recv, ccw_send, ccw_recv, copy_sems,
             w_sem):
        my = lax.axis_index("i")
        left = lax.rem(my + N_DEV - 1, N_DEV)
        right = lax.rem(my + 1, N_DEV)

        barrier_sem = pltpu.get_barrier_semaphore()
        for nbr in (left, right):
            pl.semaphore_signal(
                barrier_sem, inc=1,
                device_id=(nbr,), device_id_type=pl.DeviceIdType.MESH,
            )

        def x_row(j):
            return (j // 2) * rows + (half if j % 2 else 0)

        cps = [
            pltpu.make_async_copy(
                x_hbm.at[pl.ds(x_row(j), rows)],
                stage.at[j % 2],
                copy_sems.at[j],
            )
            for j in range(2 * SUB)
        ]
        cpw = pltpu.make_async_copy(w_hbm, w_f32, w_sem)
        cpw.start()
        cps[0].start()
        cps[1].start()

        pl.semaphore_wait(barrier_sem, 2)

        pending = []
        cw_rdmas = {}
        ccw_rdmas = {}

        def rsend(is_cw, h, s, src):
            ref, sems_s, sems_r, nbr = (
                (cw_ref, cw_send, cw_recv, right) if is_cw
                else (ccw_ref, ccw_send, ccw_recv, left)
            )
            r = pltpu.make_async_remote_copy(
                src_ref=src,
                dst_ref=ref.at[h].at[pl.ds(s * rows, rows)],
                send_sem=sems_s.at[h, s],
                recv_sem=sems_r.at[h, s],
                device_id=(nbr,),
                device_id_type=pl.DeviceIdType.MESH,
            )
            r.start()
            pending.append(r)
            (cw_rdmas if is_cw else ccw_rdmas)[(h, s)] = r

        def gelu_store(chunk, row_start, nrows):
            y = lax.dot_general(
                chunk, w_ref[...],
                (((1,), (0,)), ((), ())),
                preferred_element_type=jnp.float32,
            )
            c = 0.7978845608028654
            g = 0.5 * y * (1.0 + jnp.tanh(c * (y + 0.044715 * y * y * y)))
            out_ref[pl.ds(row_start, nrows), :] = g

        def cw_origin(j):
            return lax.rem(my + N_DEV - j - 1, N_DEV)

        def ccw_origin(j):
            return lax.rem(my + j + 1, N_DEV)

        for j in range(2 * SUB):
            cps[j].wait()
            r0 = x_row(j)
            x_bf[pl.ds(r0, rows)] = stage[j % 2].astype(jnp.bfloat16)
            if j + 2 < 2 * SUB:
                cps[j + 2].start()
            rsend(j % 2 == 0, 0, j // 2, x_bf.at[pl.ds(r0, rows)])

        cpw.wait()
        w_ref[...] = w_f32[...].astype(jnp.bfloat16)
        gelu_store(x_bf[...], my * m_per, m_per)

        for h in range(1, N_HOP):
            for s in range(SUB):
                cw_rdmas[(h - 1, s)].wait_recv()
                rsend(True, h, s, cw_ref.at[h - 1].at[pl.ds(s * rows, rows)])
                ccw_rdmas[(h - 1, s)].wait_recv()
                rsend(False, h, s, ccw_ref.at[h - 1].at[pl.ds(s * rows, rows)])
            gelu_store(cw_ref[h - 1], cw_origin(h - 1) * m_per, half)
            gelu_store(ccw_ref[h - 1], ccw_origin(h - 1) * m_per + half, half)

        co = cw_origin(N_HOP - 1) * m_per
        xo = ccw_origin(N_HOP - 1) * m_per + half
        for s in range(SUB):
            cw_rdmas[(N_HOP - 1, s)].wait_recv()
            gelu_store(cw_ref[N_HOP - 1, pl.ds(s * rows, rows)],
                       co + s * rows, rows)
            ccw_rdmas[(N_HOP - 1, s)].wait_recv()
            gelu_store(ccw_ref[N_HOP - 1, pl.ds(s * rows, rows)],
                       xo + s * rows, rows)

        for r in pending:
            r.wait_send()

    return pl.pallas_call(
        body,
        out_shape=jax.ShapeDtypeStruct((N_DEV * m_per, n), jnp.float32),
        in_specs=[
            pl.BlockSpec(memory_space=pl.ANY),
            pl.BlockSpec(memory_space=pl.ANY),
        ],
        out_specs=pl.BlockSpec(memory_space=pltpu.VMEM),
        scratch_shapes=[
            pltpu.VMEM((2, rows, k), jnp.float32),
            pltpu.VMEM((k, n), jnp.float32),
            pltpu.VMEM((k, n), jnp.bfloat16),
            pltpu.VMEM((m_per, k), jnp.bfloat16),
            pltpu.VMEM((N_HOP, half, k), jnp.bfloat16),
            pltpu.VMEM((N_HOP, half, k), jnp.bfloat16),
            pltpu.SemaphoreType.DMA((N_HOP, SUB)),
            pltpu.SemaphoreType.DMA((N_HOP, SUB)),
            pltpu.SemaphoreType.DMA((N_HOP, SUB)),
            pltpu.SemaphoreType.DMA((N_HOP, SUB)),
            pltpu.SemaphoreType.DMA((2 * SUB,)),
            pltpu.SemaphoreType.DMA,
        ],
        compiler_params=pltpu.CompilerParams(
            collective_id=0,
            vmem_limit_bytes=100 * 1024 * 1024,
        ),
    )(x, w_mat)


# device time: 154348 ns/iter; 1.1157x vs baseline; 1.0007x over previous
import jax
import jax.numpy as jnp
from jax import lax
from jax.experimental import pallas as pl
from jax.experimental.pallas import tpu as pltpu

N_DEV = 4
N_HOP = N_DEV - 1
SUB = 8


def kernel(x, w_mat):
    m_per, k = x.shape
    _, n = w_mat.shape
    half = m_per // 2
    rows = half // SUB

    def body(x_hbm, w_hbm, out_ref, stage, w_f32, w_ref, x_bf, cw_ref,
             ccw_ref, cw_send, cw_recv, ccw_send, ccw_recv, copy_sems,
             w_sem):
        my = lax.axis_index("i")
        left = lax.rem(my + N_DEV - 1, N_DEV)
        right = lax.rem(my + 1, N_DEV)

        barrier_sem = pltpu.get_barrier_semaphore()
        for nbr in (left, right):
            pl.semaphore_signal(
                barrier_sem, inc=1,
                device_id=(nbr,), device_id_type=pl.DeviceIdType.MESH,
            )

        def x_row(j):
            return (j // 2) * rows + (half if j % 2 else 0)

        cps = [
            pltpu.make_async_copy(
                x_hbm.at[pl.ds(x_row(j), rows)],
                stage.at[j % 2],
                copy_sems.at[j],
            )
            for j in range(2 * SUB)
        ]
        cpw = pltpu.make_async_copy(w_hbm, w_f32, w_sem)
        cpw.start()
        cps[0].start()
        cps[1].start()

        pl.semaphore_wait(barrier_sem, 2)

        pending = []
        cw_rdmas = {}
        ccw_rdmas = {}

        def rsend(is_cw, h, s, src):
            ref, sems_s, sems_r, nbr = (
                (cw_ref, cw_send, cw_recv, right) if is_cw
                else (ccw_ref, ccw_send, ccw_recv, left)
            )
            r = pltpu.make_async_remote_copy(
                src_ref=src,
                dst_ref=ref.at[h].at[pl.ds(s * rows, rows)],
                send_sem=sems_s.at[h, s],
                recv_sem=sems_r.at[h, s],
                device_id=(nbr,),
                device_id_type=pl.DeviceIdType.MESH,
            )
            r.start()
            pending.append(r)
            (cw_rdmas if is_cw else ccw_rdmas)[(h, s)] = r

        def gelu_store(chunk, row_start, nrows):
            y = lax.dot_general(
                chunk, w_ref[...],
                (((1,), (0,)), ((), ())),
                preferred_element_type=jnp.float32,
            )
            c = 0.7978845608028654
            g = 0.5 * y * (1.0 + jnp.tanh(c * (y + 0.044715 * y * y * y)))
            out_ref[pl.ds(row_start, nrows), :] = g

        def cw_origin(j):
            return lax.rem(my + N_DEV - j - 1, N_DEV)

        def ccw_origin(j):
            return lax.rem(my + j + 1, N_DEV)

        for j in range(2 * SUB):
            cps[j].wait()
            r0 = x_row(j)
            x_bf[pl.ds(r0, rows)] = stage[j % 2].astype(jnp.bfloat16)
            if j + 2 < 2 * SUB:
                cps[j + 2].start()
            rsend(j % 2 == 0, 0, j // 2, x_bf.at[pl.ds(r0, rows)])

        cpw.wait()
        w_ref[...] = w_f32[...].astype(jnp.bfloat16)
        gelu_store(x_bf[...], my * m_per, m_per)

        for h in range(1, N_HOP):
            for s in range(SUB):
                cw_rdmas[(h - 1, s)].wait_recv()
                rsend(True, h, s, cw_ref.at[h - 1].at[pl.ds(s * rows, rows)])
                ccw_rdmas[(h - 1, s)].wait_recv()
                rsend(False, h, s, ccw_ref.at[h - 1].at[pl.ds(s * rows, rows)])
            gelu_store(cw_ref[h - 1], cw_origin(h - 1) * m_per, half)
            gelu_store(ccw_ref[h - 1], ccw_origin(h - 1) * m_per + half, half)

        co = cw_origin(N_HOP - 1) * m_per
        xo = ccw_origin(N_HOP - 1) * m_per + half
        for s in range(SUB):
            cw_rdmas[(N_HOP - 1, s)].wait_recv()
            gelu_store(cw_ref[N_HOP - 1, pl.ds(s * rows, rows)],
                       co + s * rows, rows)
            ccw_rdmas[(N_HOP - 1, s)].wait_recv()
            gelu_store(ccw_ref[N_HOP - 1, pl.ds(s * rows, rows)],
                       xo + s * rows, rows)

        for r in pending:
            r.wait_send()

    return pl.pallas_call(
        body,
        out_shape=jax.ShapeDtypeStruct((N_DEV * m_per, n), jnp.float32),
        in_specs=[
            pl.BlockSpec(memory_space=pl.ANY),
            pl.BlockSpec(memory_space=pl.ANY),
        ],
        out_specs=pl.BlockSpec(memory_space=pltpu.VMEM),
        scratch_shapes=[
            pltpu.VMEM((2, rows, k), jnp.float32),
            pltpu.VMEM((k, n), jnp.float32),
            pltpu.VMEM((k, n), jnp.bfloat16),
            pltpu.VMEM((m_per, k), jnp.bfloat16),
            pltpu.VMEM((N_HOP, half, k), jnp.bfloat16),
            pltpu.VMEM((N_HOP, half, k), jnp.bfloat16),
            pltpu.SemaphoreType.DMA((N_HOP, SUB)),
            pltpu.SemaphoreType.DMA((N_HOP, SUB)),
            pltpu.SemaphoreType.DMA((N_HOP, SUB)),
            pltpu.SemaphoreType.DMA((N_HOP, SUB)),
            pltpu.SemaphoreType.DMA((2 * SUB,)),
            pltpu.SemaphoreType.DMA,
        ],
        compiler_params=pltpu.CompilerParams(
            collective_id=0,
            vmem_limit_bytes=100 * 1024 * 1024,
        ),
    )(x, w_mat)


# device time: 111753 ns/iter; 1.5410x vs baseline; 1.3812x over previous
import jax
import jax.numpy as jnp
from jax import lax
from jax.experimental import pallas as pl
from jax.experimental.pallas import tpu as pltpu

N_DEV = 4
N_HOP = N_DEV - 1
WSUB = 4
XSUB = 8

LIN_A, LIN_B, RIN_A, RIN_B, TR_L, TR_R, DIAG_L, DIAG_R = range(8)
YL_A, YL_B, YR_A, YR_B, YO_A, YO_B = range(6)
FWD_L, FWD_R = 6, 7


def kernel(x, w_mat):
    m_per, k = x.shape
    _, n = w_mat.shape
    hn = n // 2
    wk = k // WSUB
    xr = m_per // XSUB

    def body(x_hbm, w_hbm, out_ref, stage_x, stage_w, x_bf, w_bf_a, w_bf_b,
             wcw, wccw, ysend, yrecv,
             wcw_s, wcw_r, wccw_s, wccw_r, ys_s, yr_r, xcp_sems, wcp_sems):
        my = lax.axis_index("i")
        left = lax.rem(my + N_DEV - 1, N_DEV)
        right = lax.rem(my + 1, N_DEV)
        opp = lax.rem(my + 2, N_DEV)

        barrier_sem = pltpu.get_barrier_semaphore()
        for nbr in (left, right):
            pl.semaphore_signal(
                barrier_sem, inc=1,
                device_id=(nbr,), device_id_type=pl.DeviceIdType.MESH,
            )

        wcps = [
            pltpu.make_async_copy(
                w_hbm.at[pl.ds(j * wk, wk)], stage_w.at[j % 2],
                wcp_sems.at[j])
            for j in range(WSUB)
        ]
        xcps = [
            pltpu.make_async_copy(
                x_hbm.at[pl.ds(j * xr, xr)], stage_x.at[j % 2],
                xcp_sems.at[j])
            for j in range(XSUB)
        ]
        wcps[0].start()
        wcps[1].start()

        pl.semaphore_wait(barrier_sem, 2)

        pending = []
        wcw_rd = {}
        wccw_rd = {}

        def wsend(is_cw, h, s, src):
            ref, ss, rr, nbr = (
                (wcw, wcw_s, wcw_r, right) if is_cw
                else (wccw, wccw_s, wccw_r, left)
            )
            r = pltpu.make_async_remote_copy(
                src_ref=src,
                dst_ref=ref.at[h].at[pl.ds(s * wk, wk)],
                send_sem=ss.at[h, s],
                recv_sem=rr.at[h, s],
                device_id=(nbr,),
                device_id_type=pl.DeviceIdType.MESH,
            )
            r.start()
            pending.append(r)
            (wcw_rd if is_cw else wccw_rd)[(h, s)] = r

        def ysend_to(sem_i, recv_i, nbr, src):
            r = pltpu.make_async_remote_copy(
                src_ref=src,
                dst_ref=yrecv.at[recv_i],
                send_sem=ys_s.at[sem_i],
                recv_sem=yr_r.at[recv_i],
                device_id=(nbr,),
                device_id_type=pl.DeviceIdType.MESH,
            )
            r.start()
            pending.append(r)

        def yrecv_wait(i):
            r = pltpu.make_async_remote_copy(
                src_ref=yrecv.at[i],
                dst_ref=yrecv.at[i],
                send_sem=ys_s.at[0],
                recv_sem=yr_r.at[i],
                device_id=(left,),
                device_id_type=pl.DeviceIdType.MESH,
            )
            r.wait_recv()

        def gelu32(xa, wb):
            y = lax.dot_general(
                xa, wb, (((1,), (0,)), ((), ())),
                preferred_element_type=jnp.float32,
            )
            c = 0.7978845608028654
            return 0.5 * y * (1.0 + jnp.tanh(c * (y + 0.044715 * y * y * y)))

        for j in range(WSUB):
            wcps[j].wait()
            wj = stage_w[j % 2]
            w_bf_a[pl.ds(j * wk, wk)] = wj[:, 0:hn].astype(jnp.bfloat16)
            w_bf_b[pl.ds(j * wk, wk)] = wj[:, hn:n].astype(jnp.bfloat16)
            if j + 2 < WSUB:
                wcps[j + 2].start()
            wsend(True, 0, j, w_bf_a.at[pl.ds(j * wk, wk)])
            wsend(False, 0, j, w_bf_b.at[pl.ds(j * wk, wk)])
        xcps[0].start()
        xcps[1].start()

        for s in range(WSUB):
            wcw_rd[(0, s)].wait_recv()
            wsend(True, 1, s, wcw.at[0].at[pl.ds(s * wk, wk)])
            wccw_rd[(0, s)].wait_recv()
            wsend(False, 1, s, wccw.at[0].at[pl.ds(s * wk, wk)])
            for j in (2 * s, 2 * s + 1):
                xcps[j].wait()
                x_bf[pl.ds(j * xr, xr)] = stage_x[j % 2].astype(jnp.bfloat16)
                if j + 2 < XSUB:
                    xcps[j + 2].start()

        out_ref[pl.ds(my * m_per, m_per), pl.ds(0, hn)] = (
            gelu32(x_bf[...], w_bf_a[...]))
        out_ref[pl.ds(my * m_per, m_per), pl.ds(hn, hn)] = (
            gelu32(x_bf[...], w_bf_b[...]))

        ysend[YL_A] = gelu32(x_bf[...], wcw[0]).astype(jnp.bfloat16)
        ysend_to(YL_A, RIN_A, left, ysend.at[YL_A])
        ysend[YR_B] = gelu32(x_bf[...], wccw[0]).astype(jnp.bfloat16)
        ysend_to(YR_B, LIN_B, right, ysend.at[YR_B])

        for s in range(WSUB):
            wcw_rd[(1, s)].wait_recv()
            wsend(True, 2, s, wcw.at[1].at[pl.ds(s * wk, wk)])
            wccw_rd[(1, s)].wait_recv()
            wsend(False, 2, s, wccw.at[1].at[pl.ds(s * wk, wk)])

        ysend[YO_A] = gelu32(x_bf[...], wcw[1]).astype(jnp.bfloat16)
        ysend_to(YO_A, TR_L, right, ysend.at[YO_A])
        ysend[YO_B] = gelu32(x_bf[...], wccw[1]).astype(jnp.bfloat16)
        ysend_to(YO_B, TR_R, left, ysend.at[YO_B])

        yrecv_wait(TR_L)
        ysend_to(FWD_L, DIAG_L, right, yrecv.at[TR_L])
        yrecv_wait(TR_R)
        ysend_to(FWD_R, DIAG_R, left, yrecv.at[TR_R])

        for s in range(WSUB):
            wcw_rd[(2, s)].wait_recv()
            wccw_rd[(2, s)].wait_recv()
        ysend[YR_A] = gelu32(x_bf[...], wcw[2]).astype(jnp.bfloat16)
        ysend_to(YR_A, LIN_A, right, ysend.at[YR_A])
        ysend[YL_B] = gelu32(x_bf[...], wccw[2]).astype(jnp.bfloat16)
        ysend_to(YL_B, RIN_B, left, ysend.at[YL_B])

        def store(i, rorigin, col0):
            yrecv_wait(i)
            out_ref[pl.ds(rorigin * m_per, m_per), pl.ds(col0, hn)] = (
                yrecv[i].astype(jnp.float32))

        store(LIN_B, left, hn)
        store(RIN_A, right, 0)
        store(DIAG_L, opp, 0)
        store(DIAG_R, opp, hn)
        store(LIN_A, left, 0)
        store(RIN_B, right, hn)

        for r in pending:
            r.wait_send()

    return pl.pallas_call(
        body,
        out_shape=jax.ShapeDtypeStruct((N_DEV * m_per, n), jnp.float32),
        in_specs=[
            pl.BlockSpec(memory_space=pl.ANY),
            pl.BlockSpec(memory_space=pl.ANY),
        ],
        out_specs=pl.BlockSpec(memory_space=pltpu.VMEM),
        scratch_shapes=[
            pltpu.VMEM((2, xr, k), jnp.float32),
            pltpu.VMEM((2, wk, n), jnp.float32),
            pltpu.VMEM((m_per, k), jnp.bfloat16),
            pltpu.VMEM((k, hn), jnp.bfloat16),
            pltpu.VMEM((k, hn), jnp.bfloat16),
            pltpu.VMEM((N_HOP, k, hn), jnp.bfloat16),
            pltpu.VMEM((N_HOP, k, hn), jnp.bfloat16),
            pltpu.VMEM((6, m_per, hn), jnp.bfloat16),
            pltpu.VMEM((8, m_per, hn), jnp.bfloat16),
            pltpu.SemaphoreType.DMA((N_HOP, WSUB)),
            pltpu.SemaphoreType.DMA((N_HOP, WSUB)),
            pltpu.SemaphoreType.DMA((N_HOP, WSUB)),
            pltpu.SemaphoreType.DMA((N_HOP, WSUB)),
            pltpu.SemaphoreType.DMA((8,)),
            pltpu.SemaphoreType.DMA((8,)),
            pltpu.SemaphoreType.DMA((XSUB,)),
            pltpu.SemaphoreType.DMA((WSUB,)),
        ],
        compiler_params=pltpu.CompilerParams(
            collective_id=0,
            vmem_limit_bytes=100 * 1024 * 1024,
        ),
    )(x, w_mat)
